# Initial kernel scaffold; baseline (speedup 1.0000x reference)
import jax
import jax.numpy as jnp
from jax.experimental import pallas as pl


def kernel(input_ids, embedding_weight):
    raise NotImplementedError("write your pallas kernel here")


# speedup vs baseline: 1.4730x; 1.4730x over previous
import functools

import jax
import jax.numpy as jnp
from jax import lax
from jax.experimental import pallas as pl
from jax.experimental.pallas import tpu as pltpu
from jax.experimental.pallas import tpu_sc as plsc

BATCH = 4
SEQ_LEN = 2048
VOCAB = 50257
HIDDEN = 768

_info = plsc.get_sparse_core_info()
_NC = _info.num_cores
_NS = _info.num_subcores
NW = _NC * _NS
N = BATCH * SEQ_LEN
PER_W = N // NW
CHUNK = 64
NCHUNK = PER_W // CHUNK

_mesh = plsc.VectorSubcoreMesh(core_axis_name="c", subcore_axis_name="s")


@functools.partial(
    pl.kernel,
    mesh=_mesh,
    out_type=jax.ShapeDtypeStruct((N, HIDDEN), jnp.float32),
    scratch_types=[
        pltpu.VMEM((NCHUNK, CHUNK), jnp.int32),
        pltpu.VMEM((CHUNK, HIDDEN), jnp.float32),
        pltpu.VMEM((CHUNK, HIDDEN), jnp.float32),
        pltpu.SemaphoreType.DMA,
        pltpu.SemaphoreType.DMA,
    ],
)
def _emb_lookup(ids_hbm, table_hbm, out_hbm, idx_v, buf0, buf1, sem0, sem1):
    wid = lax.axis_index("s") * _NC + lax.axis_index("c")
    base = wid * PER_W
    pltpu.sync_copy(ids_hbm.at[wid], idx_v)

    bufs = (buf0, buf1)
    sems = (sem0, sem1)
    handles = [None] * NCHUNK
    handles[0] = pltpu.async_copy(table_hbm.at[idx_v.at[0]], buf0, sem0)
    if NCHUNK > 1:
        handles[1] = pltpu.async_copy(table_hbm.at[idx_v.at[1]], buf1, sem1)
    for c in range(NCHUNK):
        handles[c].wait()
        pltpu.sync_copy(bufs[c % 2], out_hbm.at[pl.ds(base + c * CHUNK, CHUNK)])
        if c + 2 < NCHUNK:
            handles[c + 2] = pltpu.async_copy(
                table_hbm.at[idx_v.at[c + 2]], bufs[c % 2], sems[c % 2]
            )


def kernel(input_ids, embedding_weight):
    ids = input_ids.astype(jnp.int32).reshape(NW, NCHUNK, CHUNK)
    out = _emb_lookup(ids, embedding_weight)
    return out.reshape(BATCH, SEQ_LEN, HIDDEN)


# speedup vs baseline: 1.5146x; 1.0283x over previous
import functools

import jax
import jax.numpy as jnp
from jax import lax
from jax.experimental import pallas as pl
from jax.experimental.pallas import tpu as pltpu
from jax.experimental.pallas import tpu_sc as plsc

BATCH = 4
SEQ_LEN = 2048
VOCAB = 50257
HIDDEN = 768

_info = plsc.get_sparse_core_info()
_NC = _info.num_cores
_NS = _info.num_subcores
NW = _NC * _NS
N = BATCH * SEQ_LEN
PER_W = N // NW
CHUNK = 32
NCHUNK = PER_W // CHUNK
NBUF = 4

_mesh = plsc.VectorSubcoreMesh(core_axis_name="c", subcore_axis_name="s")


@functools.partial(
    pl.kernel,
    mesh=_mesh,
    out_type=jax.ShapeDtypeStruct((N, HIDDEN), jnp.float32),
    scratch_types=[
        pltpu.VMEM((NCHUNK, CHUNK), jnp.int32),
        *[pltpu.VMEM((CHUNK, HIDDEN), jnp.float32) for _ in range(NBUF)],
        *[pltpu.SemaphoreType.DMA for _ in range(2 * NBUF)],
    ],
)
def _emb_lookup(ids_hbm, table_hbm, out_hbm, idx_v, *rest):
    bufs = rest[:NBUF]
    gsems = rest[NBUF : 2 * NBUF]
    ssems = rest[2 * NBUF : 3 * NBUF]
    wid = lax.axis_index("s") * _NC + lax.axis_index("c")
    base = wid * PER_W
    pltpu.sync_copy(ids_hbm.at[wid], idx_v)

    g = [None] * NCHUNK
    s = [None] * NCHUNK
    for c in range(NBUF):
        g[c] = pltpu.async_copy(table_hbm.at[idx_v.at[c]], bufs[c], gsems[c])
    for c in range(NCHUNK):
        b = c % NBUF
        g[c].wait()
        s[c] = pltpu.async_copy(
            bufs[b], out_hbm.at[pl.ds(base + c * CHUNK, CHUNK)], ssems[b]
        )
        if c + NBUF < NCHUNK:
            s[c].wait()
            g[c + NBUF] = pltpu.async_copy(
                table_hbm.at[idx_v.at[c + NBUF]], bufs[b], gsems[b]
            )
    for c in range(NCHUNK - NBUF, NCHUNK):
        if c >= 0:
            s[c].wait()


def kernel(input_ids, embedding_weight):
    ids = input_ids.astype(jnp.int32).reshape(NW, NCHUNK, CHUNK)
    out = _emb_lookup(ids, embedding_weight)
    return out.reshape(BATCH, SEQ_LEN, HIDDEN)


# speedup vs baseline: 1.5149x; 1.0002x over previous
import functools

import jax
import jax.numpy as jnp
from jax import lax
from jax.experimental import pallas as pl
from jax.experimental.pallas import tpu as pltpu
from jax.experimental.pallas import tpu_sc as plsc

BATCH = 4
SEQ_LEN = 2048
VOCAB = 50257
HIDDEN = 768

_info = plsc.get_sparse_core_info()
_NC = _info.num_cores
_NS = _info.num_subcores
NW = _NC * _NS
N = BATCH * SEQ_LEN
PER_W = N // NW
CHUNK = 32
NCHUNK = PER_W // CHUNK
NBUF = 4

_mesh = plsc.VectorSubcoreMesh(core_axis_name="c", subcore_axis_name="s")


W_PER_B = SEQ_LEN // PER_W


@functools.partial(
    pl.kernel,
    mesh=_mesh,
    out_type=jax.ShapeDtypeStruct((BATCH, SEQ_LEN, HIDDEN), jnp.float32),
    scratch_types=[
        pltpu.VMEM((PER_W,), jnp.int32),
        *[pltpu.VMEM((CHUNK, HIDDEN), jnp.float32) for _ in range(NBUF)],
        *[pltpu.SemaphoreType.DMA for _ in range(2 * NBUF)],
    ],
)
def _emb_lookup(ids_hbm, table_hbm, out_hbm, idx_v, *rest):
    bufs = rest[:NBUF]
    gsems = rest[NBUF : 2 * NBUF]
    ssems = rest[2 * NBUF : 3 * NBUF]
    wid = lax.axis_index("s") * _NC + lax.axis_index("c")
    row = wid // W_PER_B
    col = (wid % W_PER_B) * PER_W
    pltpu.sync_copy(ids_hbm.at[row, pl.ds(col, PER_W)], idx_v)

    g = [None] * NCHUNK
    s = [None] * NCHUNK

    def gather(c, b):
        return pltpu.async_copy(
            table_hbm.at[idx_v.at[pl.ds(c * CHUNK, CHUNK)]], bufs[b], gsems[b]
        )

    for c in range(NBUF):
        g[c] = gather(c, c)
    for c in range(NCHUNK):
        b = c % NBUF
        g[c].wait()
        s[c] = pltpu.async_copy(
            bufs[b], out_hbm.at[row, pl.ds(col + c * CHUNK, CHUNK)], ssems[b]
        )
        if c + NBUF < NCHUNK:
            s[c].wait()
            g[c + NBUF] = gather(c + NBUF, b)
    for c in range(max(0, NCHUNK - NBUF), NCHUNK):
        s[c].wait()


def kernel(input_ids, embedding_weight):
    return _emb_lookup(input_ids.astype(jnp.int32), embedding_weight)
